# Initial kernel scaffold; baseline (speedup 1.0000x reference)
import jax
import jax.numpy as jnp
from jax.experimental import pallas as pl


def kernel(x, table, W1, W2):
    raise NotImplementedError("write your pallas kernel here")


# speedup vs baseline: 1.2608x; 1.2608x over previous
import functools

import jax
import jax.numpy as jnp
from jax import lax
from jax.experimental import pallas as pl
from jax.experimental.pallas import tpu as pltpu
from jax.experimental.pallas import tpu_sc as plsc

N_LEVELS = 16
F = 2
LOG2_T = 19
T = 1 << LOG2_T
BASE_RES = 16
PRIME_Y = 2654435761
PRIME_Z = 805459861
DENSE_LEVELS = 3

N_POINTS = 262144
ENC_DIM = N_LEVELS * F
N_NEURONS = 64
OUT_DIMS = 16

NC = 2
NS = 16
NW = NC * NS
PW = N_POINTS // NW
C = 1024
NG = C // 16
NCH = PW // C

_MESH = plsc.VectorSubcoreMesh(
    core_axis_name="c", subcore_axis_name="s", num_cores=NC, num_subcores=NS
)

_ENC_SCRATCH = (
    [pltpu.VMEM((C,), jnp.float32) for _ in range(6)]
    + [pltpu.VMEM((C,), jnp.int32) for _ in range(8)]
    + [pltpu.VMEM((C,), jnp.float32) for _ in range(8)]
    + [pltpu.VMEM((C,), jnp.float32) for _ in range(8)]
    + [pltpu.VMEM((ENC_DIM, C), jnp.float32)]
    + [pltpu.SemaphoreType.DMA]
)


@functools.partial(
    pl.kernel,
    out_type=jax.ShapeDtypeStruct((ENC_DIM, N_POINTS), jnp.float32),
    mesh=_MESH,
    scratch_types=_ENC_SCRATCH,
)
def _encode(xs_hbm, ys_hbm, zs_hbm, tab0_hbm, tab1_hbm, out_hbm,
            xsv, ysv, zsv, fxv, fyv, fzv,
            i0, i1, i2, i3, i4, i5, i6, i7,
            a0, a1, a2, a3, a4, a5, a6, a7,
            b0, b1, b2, b3, b4, b5, b6, b7,
            encb, sem):
    idx_refs = (i0, i1, i2, i3, i4, i5, i6, i7)
    f0_refs = (a0, a1, a2, a3, a4, a5, a6, a7)
    f1_refs = (b0, b1, b2, b3, b4, b5, b6, b7)
    wid = lax.axis_index("s") * NC + lax.axis_index("c")

    def process_level(scale, loff, lrow, res):
        def p1_body(g, cr):
            b = g * 16
            px = xsv[pl.ds(b, 16)] * scale + 0.5
            py = ysv[pl.ds(b, 16)] * scale + 0.5
            pz = zsv[pl.ds(b, 16)] * scale + 0.5
            gx = px.astype(jnp.uint32)
            gy = py.astype(jnp.uint32)
            gz = pz.astype(jnp.uint32)
            fxv[pl.ds(b, 16)] = px - gx.astype(jnp.float32)
            fyv[pl.ds(b, 16)] = py - gy.astype(jnp.float32)
            fzv[pl.ds(b, 16)] = pz - gz.astype(jnp.float32)
            for c in range(8):
                ox, oy, oz = c & 1, (c >> 1) & 1, (c >> 2) & 1
                cx = gx + jnp.uint32(ox) if ox else gx
                cy = gy + jnp.uint32(oy) if oy else gy
                cz = gz + jnp.uint32(oz) if oz else gz
                if res is not None:
                    h = cx + cy * jnp.uint32(res) + cz * jnp.uint32(res * res)
                else:
                    h = (cx ^ (cy * jnp.uint32(PRIME_Y))
                         ^ (cz * jnp.uint32(PRIME_Z))) & jnp.uint32(T - 1)
                idx_refs[c][pl.ds(b, 16)] = h.astype(jnp.int32) + loff
            return cr

        lax.fori_loop(0, NG, p1_body, 0)

        copies = [pltpu.async_copy(tab0_hbm.at[idx_refs[c]], f0_refs[c], sem)
                  for c in range(8)]
        copies += [pltpu.async_copy(tab1_hbm.at[idx_refs[c]], f1_refs[c], sem)
                   for c in range(8)]
        for cp in copies:
            cp.wait()

        def p2_body(g, cr):
            b = g * 16
            fx = fxv[pl.ds(b, 16)]
            fy = fyv[pl.ds(b, 16)]
            fz = fzv[pl.ds(b, 16)]
            ux = 1.0 - fx
            uy = 1.0 - fy
            uz = 1.0 - fz
            acc0 = jnp.zeros((16,), jnp.float32)
            acc1 = jnp.zeros((16,), jnp.float32)
            for c in range(8):
                ox, oy, oz = c & 1, (c >> 1) & 1, (c >> 2) & 1
                w = ((fx if ox else ux) * (fy if oy else uy)
                     * (fz if oz else uz))
                acc0 = acc0 + w * f0_refs[c][pl.ds(b, 16)]
                acc1 = acc1 + w * f1_refs[c][pl.ds(b, 16)]
            encb[lrow, pl.ds(b, 16)] = acc0
            encb[lrow + 1, pl.ds(b, 16)] = acc1
            return cr

        lax.fori_loop(0, NG, p2_body, 0)

    def chunk_body(ch, cr):
        pbase = wid * PW + ch * C
        pltpu.sync_copy(xs_hbm.at[pl.ds(pbase, C)], xsv)
        pltpu.sync_copy(ys_hbm.at[pl.ds(pbase, C)], ysv)
        pltpu.sync_copy(zs_hbm.at[pl.ds(pbase, C)], zsv)
        for l in range(DENSE_LEVELS):
            res = BASE_RES * (2 ** l)
            process_level(float(res - 1), l * T, 2 * l, res)

        def lvl_body(l, lcr):
            scale = (jnp.int32(1) << (l + 4)).astype(jnp.float32) - 1.0
            process_level(scale, l * T, 2 * l, None)
            return lcr

        lax.fori_loop(DENSE_LEVELS, N_LEVELS, lvl_body, 0)
        pltpu.sync_copy(encb, out_hbm.at[:, pl.ds(pbase, C)])
        return cr

    lax.fori_loop(0, NCH, chunk_body, 0)


BN = 2048


def _mlp_body(e_ref, w1_ref, w2_ref, o_ref):
    h = lax.dot_general(w1_ref[...], e_ref[...], (((0,), (0,)), ((), ())),
                        preferred_element_type=jnp.float32)
    h = jnp.maximum(h, 0.0)
    o_ref[...] = lax.dot_general(h, w2_ref[...], (((0,), (0,)), ((), ())),
                                 preferred_element_type=jnp.float32)


def _mlp(enc_t, W1, W2):
    return pl.pallas_call(
        _mlp_body,
        grid=(N_POINTS // BN,),
        in_specs=[
            pl.BlockSpec((ENC_DIM, BN), lambda i: (0, i)),
            pl.BlockSpec((ENC_DIM, N_NEURONS), lambda i: (0, 0)),
            pl.BlockSpec((N_NEURONS, OUT_DIMS), lambda i: (0, 0)),
        ],
        out_specs=pl.BlockSpec((BN, OUT_DIMS), lambda i: (i, 0)),
        out_shape=jax.ShapeDtypeStruct((N_POINTS, OUT_DIMS), jnp.float32),
    )(enc_t, W1, W2)


def kernel(x, table, W1, W2):
    xs = x[:, 0]
    ys = x[:, 1]
    zs = x[:, 2]
    tab0 = table[:, :, 0].reshape(N_LEVELS * T)
    tab1 = table[:, :, 1].reshape(N_LEVELS * T)
    enc_t = _encode(xs, ys, zs, tab0, tab1)
    return _mlp(enc_t, W1, W2)


# speedup vs baseline: 1.9261x; 1.5277x over previous
import functools

import jax
import jax.numpy as jnp
from jax import lax
from jax.experimental import pallas as pl
from jax.experimental.pallas import tpu as pltpu
from jax.experimental.pallas import tpu_sc as plsc

N_LEVELS = 16
F = 2
LOG2_T = 19
T = 1 << LOG2_T
BASE_RES = 16
PRIME_Y = 2654435761
PRIME_Z = 805459861
DENSE_LEVELS = 3

N_POINTS = 262144
ENC_DIM = N_LEVELS * F
N_NEURONS = 64
OUT_DIMS = 16

NC = 2
NS = 16
NW = NC * NS
PW = N_POINTS // NW
C = 1024
NG = C // 16
NCH = PW // C

_MESH = plsc.VectorSubcoreMesh(
    core_axis_name="c", subcore_axis_name="s", num_cores=NC, num_subcores=NS
)

_ENC_SCRATCH = (
    [pltpu.VMEM((C,), jnp.float32) for _ in range(6)]
    + [pltpu.VMEM((C,), jnp.int32) for _ in range(8)]
    + [pltpu.VMEM((C,), jnp.int32) for _ in range(8)]
    + [pltpu.VMEM((ENC_DIM, C), jnp.float32)]
    + [pltpu.SemaphoreType.DMA]
)


@functools.partial(
    pl.kernel,
    out_type=jax.ShapeDtypeStruct((ENC_DIM, N_POINTS), jnp.float32),
    mesh=_MESH,
    scratch_types=_ENC_SCRATCH,
)
def _encode(xs_hbm, ys_hbm, zs_hbm, tab_hbm, out_hbm,
            xsv, ysv, zsv, fxv, fyv, fzv,
            i0, i1, i2, i3, i4, i5, i6, i7,
            a0, a1, a2, a3, a4, a5, a6, a7,
            encb, sem):
    idx_refs = (i0, i1, i2, i3, i4, i5, i6, i7)
    row_refs = (a0, a1, a2, a3, a4, a5, a6, a7)
    wid = lax.axis_index("s") * NC + lax.axis_index("c")

    def process_level(scale, loff, lrow, res):
        def p1_body(g, cr):
            b = g * 16
            px = xsv[pl.ds(b, 16)] * scale + 0.5
            py = ysv[pl.ds(b, 16)] * scale + 0.5
            pz = zsv[pl.ds(b, 16)] * scale + 0.5
            gx = px.astype(jnp.uint32)
            gy = py.astype(jnp.uint32)
            gz = pz.astype(jnp.uint32)
            fxv[pl.ds(b, 16)] = px - gx.astype(jnp.float32)
            fyv[pl.ds(b, 16)] = py - gy.astype(jnp.float32)
            fzv[pl.ds(b, 16)] = pz - gz.astype(jnp.float32)
            for c in range(8):
                ox, oy, oz = c & 1, (c >> 1) & 1, (c >> 2) & 1
                cx = gx + jnp.uint32(ox) if ox else gx
                cy = gy + jnp.uint32(oy) if oy else gy
                cz = gz + jnp.uint32(oz) if oz else gz
                if res is not None:
                    h = cx + cy * jnp.uint32(res) + cz * jnp.uint32(res * res)
                else:
                    h = (cx ^ (cy * jnp.uint32(PRIME_Y))
                         ^ (cz * jnp.uint32(PRIME_Z))) & jnp.uint32(T - 1)
                idx_refs[c][pl.ds(b, 16)] = h.astype(jnp.int32) + loff
            return cr

        lax.fori_loop(0, NG, p1_body, 0)

        copies = [pltpu.async_copy(tab_hbm.at[idx_refs[c]], row_refs[c], sem)
                  for c in range(8)]
        for cp in copies:
            cp.wait()

        def p2_body(g, cr):
            b = g * 16
            fx = fxv[pl.ds(b, 16)]
            fy = fyv[pl.ds(b, 16)]
            fz = fzv[pl.ds(b, 16)]
            ux = 1.0 - fx
            uy = 1.0 - fy
            uz = 1.0 - fz
            acc0 = jnp.zeros((16,), jnp.float32)
            acc1 = jnp.zeros((16,), jnp.float32)
            for c in range(8):
                ox, oy, oz = c & 1, (c >> 1) & 1, (c >> 2) & 1
                w = ((fx if ox else ux) * (fy if oy else uy)
                     * (fz if oz else uz))
                vu = row_refs[c][pl.ds(b, 16)].astype(jnp.uint32)
                f0 = lax.bitcast_convert_type(vu << jnp.uint32(16),
                                              jnp.float32)
                f1 = lax.bitcast_convert_type(vu & jnp.uint32(0xFFFF0000),
                                              jnp.float32)
                acc0 = acc0 + w * f0
                acc1 = acc1 + w * f1
            encb[lrow, pl.ds(b, 16)] = acc0
            encb[lrow + 1, pl.ds(b, 16)] = acc1
            return cr

        lax.fori_loop(0, NG, p2_body, 0)

    def chunk_body(ch, cr):
        pbase = wid * PW + ch * C
        pltpu.sync_copy(xs_hbm.at[pl.ds(pbase, C)], xsv)
        pltpu.sync_copy(ys_hbm.at[pl.ds(pbase, C)], ysv)
        pltpu.sync_copy(zs_hbm.at[pl.ds(pbase, C)], zsv)
        for l in range(DENSE_LEVELS):
            res = BASE_RES * (2 ** l)
            process_level(float(res - 1), l * T, 2 * l, res)

        def lvl_body(l, lcr):
            scale = (jnp.int32(1) << (l + 4)).astype(jnp.float32) - 1.0
            process_level(scale, l * T, 2 * l, None)
            return lcr

        lax.fori_loop(DENSE_LEVELS, N_LEVELS, lvl_body, 0)
        pltpu.sync_copy(encb, out_hbm.at[:, pl.ds(pbase, C)])
        return cr

    lax.fori_loop(0, NCH, chunk_body, 0)


BN = 2048


def _mlp_body(e_ref, w1_ref, w2_ref, o_ref):
    h = lax.dot_general(w1_ref[...], e_ref[...], (((0,), (0,)), ((), ())),
                        preferred_element_type=jnp.float32)
    h = jnp.maximum(h, 0.0)
    o_ref[...] = lax.dot_general(h, w2_ref[...], (((0,), (0,)), ((), ())),
                                 preferred_element_type=jnp.float32)


def _mlp(enc_t, W1, W2):
    return pl.pallas_call(
        _mlp_body,
        grid=(N_POINTS // BN,),
        in_specs=[
            pl.BlockSpec((ENC_DIM, BN), lambda i: (0, i)),
            pl.BlockSpec((ENC_DIM, N_NEURONS), lambda i: (0, 0)),
            pl.BlockSpec((N_NEURONS, OUT_DIMS), lambda i: (0, 0)),
        ],
        out_specs=pl.BlockSpec((BN, OUT_DIMS), lambda i: (i, 0)),
        out_shape=jax.ShapeDtypeStruct((N_POINTS, OUT_DIMS), jnp.float32),
    )(enc_t, W1, W2)


def kernel(x, table, W1, W2):
    xs = x[:, 0]
    ys = x[:, 1]
    zs = x[:, 2]
    tabp = lax.bitcast_convert_type(
        table.astype(jnp.bfloat16), jnp.int32).reshape(N_LEVELS * T)
    enc_t = _encode(xs, ys, zs, tabp)
    return _mlp(enc_t, W1, W2)


# speedup vs baseline: 2.9838x; 1.5491x over previous
import functools

import jax
import jax.numpy as jnp
from jax import lax
from jax.experimental import pallas as pl
from jax.experimental.pallas import tpu as pltpu
from jax.experimental.pallas import tpu_sc as plsc

N_LEVELS = 16
F = 2
LOG2_T = 19
T = 1 << LOG2_T
BASE_RES = 16
PRIME_Y = 2654435761
PRIME_Z = 805459861

N_POINTS = 262144
ENC_DIM = N_LEVELS * F
N_NEURONS = 64
OUT_DIMS = 16

NC = 2
NS = 16
NW = NC * NS
PW = N_POINTS // NW
C = 1024
NG = C // 16
NCH = PW // C

T0_SIZE = 4480
T1_SIZE = 33920

_MESH = plsc.VectorSubcoreMesh(
    core_axis_name="c", subcore_axis_name="s", num_cores=NC, num_subcores=NS
)

_ENC_SCRATCH = (
    [pltpu.VMEM((C,), jnp.float32) for _ in range(3)]
    + [pltpu.VMEM((C,), jnp.float32) for _ in range(6)]
    + [pltpu.VMEM((8 * C,), jnp.int32) for _ in range(2)]
    + [pltpu.VMEM((8 * C,), jnp.float32) for _ in range(2)]
    + [pltpu.VMEM((ENC_DIM, C), jnp.float32)]
    + [pltpu.VMEM((T0_SIZE,), jnp.float32),
       pltpu.VMEM((T1_SIZE,), jnp.float32)]
    + [pltpu.SemaphoreType.DMA, pltpu.SemaphoreType.DMA]
)


def _unpack_pair(v):
    vu = lax.bitcast_convert_type(v, jnp.uint32)
    f0 = lax.bitcast_convert_type(vu << jnp.uint32(16), jnp.float32)
    f1 = lax.bitcast_convert_type(vu & jnp.uint32(0xFFFF0000), jnp.float32)
    return f0, f1


@functools.partial(
    pl.kernel,
    out_type=jax.ShapeDtypeStruct((ENC_DIM, N_POINTS), jnp.float32),
    mesh=_MESH,
    scratch_types=_ENC_SCRATCH,
    compiler_params=pltpu.CompilerParams(needs_layout_passes=False),
)
def _encode(xs_hbm, ys_hbm, zs_hbm, tab_hbm, out_hbm,
            xsv, ysv, zsv,
            fxa, fya, fza, fxb, fyb, fzb,
            idx_a, idx_b, rows_a, rows_b,
            encb, t0v, t1v, sema, semb):
    frac_a = (fxa, fya, fza)
    frac_b = (fxb, fyb, fzb)
    wid = lax.axis_index("s") * NC + lax.axis_index("c")

    pltpu.sync_copy(tab_hbm.at[pl.ds(0, T0_SIZE)], t0v)
    pltpu.sync_copy(tab_hbm.at[pl.ds(T, T1_SIZE)], t1v)

    def corner_index(gx, gy, gz, c, res):
        ox, oy, oz = c & 1, (c >> 1) & 1, (c >> 2) & 1
        cx = gx + jnp.uint32(ox) if ox else gx
        cy = gy + jnp.uint32(oy) if oy else gy
        cz = gz + jnp.uint32(oz) if oz else gz
        if res is not None:
            return cx + cy * jnp.uint32(res) + cz * jnp.uint32(res * res)
        h = cx ^ (cy * jnp.uint32(PRIME_Y)) ^ (cz * jnp.uint32(PRIME_Z))
        return h & jnp.uint32(T - 1)

    def grid_coords(b, scale):
        px = xsv[pl.ds(b, 16)] * scale + 0.5
        py = ysv[pl.ds(b, 16)] * scale + 0.5
        pz = zsv[pl.ds(b, 16)] * scale + 0.5
        gx = px.astype(jnp.uint32)
        gy = py.astype(jnp.uint32)
        gz = pz.astype(jnp.uint32)
        fx = px - gx.astype(jnp.float32)
        fy = py - gy.astype(jnp.float32)
        fz = pz - gz.astype(jnp.float32)
        return gx, gy, gz, fx, fy, fz

    def corner_weight(fx, fy, fz, ux, uy, uz, c):
        ox, oy, oz = c & 1, (c >> 1) & 1, (c >> 2) & 1
        return ((fx if ox else ux) * (fy if oy else uy)
                * (fz if oz else uz))

    def pass1(scale, loff, idxs, fracs, res):
        def body(g, cr):
            b = g * 16
            gx, gy, gz, fx, fy, fz = grid_coords(b, scale)
            fracs[0][pl.ds(b, 16)] = fx
            fracs[1][pl.ds(b, 16)] = fy
            fracs[2][pl.ds(b, 16)] = fz
            for c in range(8):
                h = corner_index(gx, gy, gz, c, res)
                idxs[pl.ds(c * C + b, 16)] = h.astype(jnp.int32) + loff
            return cr
        lax.fori_loop(0, NG, body, 0)

    def issue(idxs, rows, sem):
        pltpu.async_copy(tab_hbm.at[idxs], rows, sem)

    def wait_bank(idxs, rows, sem):
        pltpu.make_async_copy(tab_hbm.at[idxs], rows, sem).wait()

    def pass2(lrow, rows, fracs):
        def body(g, cr):
            b = g * 16
            fx = fracs[0][pl.ds(b, 16)]
            fy = fracs[1][pl.ds(b, 16)]
            fz = fracs[2][pl.ds(b, 16)]
            ux, uy, uz = 1.0 - fx, 1.0 - fy, 1.0 - fz
            acc0 = jnp.zeros((16,), jnp.float32)
            acc1 = jnp.zeros((16,), jnp.float32)
            for c in range(8):
                w = corner_weight(fx, fy, fz, ux, uy, uz, c)
                f0, f1 = _unpack_pair(rows[pl.ds(c * C + b, 16)])
                acc0 = acc0 + w * f0
                acc1 = acc1 + w * f1
            encb[lrow, pl.ds(b, 16)] = acc0
            encb[lrow + 1, pl.ds(b, 16)] = acc1
            return cr
        lax.fori_loop(0, NG, body, 0)

    def fused_local_level(lrow, res, t_ref):
        def body(g, cr):
            b = g * 16
            gx, gy, gz, fx, fy, fz = grid_coords(b, float(res - 1))
            ux, uy, uz = 1.0 - fx, 1.0 - fy, 1.0 - fz
            acc0 = jnp.zeros((16,), jnp.float32)
            acc1 = jnp.zeros((16,), jnp.float32)
            for c in range(8):
                h = corner_index(gx, gy, gz, c, res)
                w = corner_weight(fx, fy, fz, ux, uy, uz, c)
                v = plsc.load_gather(t_ref, [h.astype(jnp.int32)])
                f0, f1 = _unpack_pair(v)
                acc0 = acc0 + w * f0
                acc1 = acc1 + w * f1
            encb[lrow, pl.ds(b, 16)] = acc0
            encb[lrow + 1, pl.ds(b, 16)] = acc1
            return cr
        lax.fori_loop(0, NG, body, 0)

    def hashed_scale(l):
        return (jnp.int32(1) << (l + 4)).astype(jnp.float32) - 1.0

    def chunk_body(ch, cr):
        pbase = wid * PW + ch * C
        pltpu.sync_copy(xs_hbm.at[pl.ds(pbase, C)], xsv)
        pltpu.sync_copy(ys_hbm.at[pl.ds(pbase, C)], ysv)
        pltpu.sync_copy(zs_hbm.at[pl.ds(pbase, C)], zsv)

        pass1(63.0, 2 * T, idx_a, frac_a, 64)
        issue(idx_a, rows_a, sema)
        fused_local_level(0, 16, t0v)
        fused_local_level(2, 32, t1v)
        pass1(127.0, 3 * T, idx_b, frac_b, None)
        issue(idx_b, rows_b, semb)
        wait_bank(idx_a, rows_a, sema)
        pass2(4, rows_a, frac_a)

        def pair_body(li, pcr):
            la = 4 + 2 * li
            pass1(hashed_scale(la), la * T, idx_a, frac_a, None)
            issue(idx_a, rows_a, sema)
            wait_bank(idx_b, rows_b, semb)
            pass2(2 * la - 2, rows_b, frac_b)
            lb = la + 1
            pass1(hashed_scale(lb), lb * T, idx_b, frac_b, None)
            issue(idx_b, rows_b, semb)
            wait_bank(idx_a, rows_a, sema)
            pass2(2 * la, rows_a, frac_a)
            return pcr
        lax.fori_loop(0, 6, pair_body, 0)

        wait_bank(idx_b, rows_b, semb)
        pass2(30, rows_b, frac_b)

        pltpu.sync_copy(encb, out_hbm.at[:, pl.ds(pbase, C)])
        return cr

    lax.fori_loop(0, NCH, chunk_body, 0)


BN = 2048


def _mlp_body(e_ref, w1_ref, w2_ref, o_ref):
    h = lax.dot_general(w1_ref[...], e_ref[...], (((0,), (0,)), ((), ())),
                        preferred_element_type=jnp.float32)
    h = jnp.maximum(h, 0.0)
    o_ref[...] = lax.dot_general(h, w2_ref[...], (((0,), (0,)), ((), ())),
                                 preferred_element_type=jnp.float32)


def _mlp(enc_t, W1, W2):
    return pl.pallas_call(
        _mlp_body,
        grid=(N_POINTS // BN,),
        in_specs=[
            pl.BlockSpec((ENC_DIM, BN), lambda i: (0, i)),
            pl.BlockSpec((ENC_DIM, N_NEURONS), lambda i: (0, 0)),
            pl.BlockSpec((N_NEURONS, OUT_DIMS), lambda i: (0, 0)),
        ],
        out_specs=pl.BlockSpec((BN, OUT_DIMS), lambda i: (i, 0)),
        out_shape=jax.ShapeDtypeStruct((N_POINTS, OUT_DIMS), jnp.float32),
    )(enc_t, W1, W2)


def kernel(x, table, W1, W2):
    xs = x[:, 0]
    ys = x[:, 1]
    zs = x[:, 2]
    tabp = lax.bitcast_convert_type(
        table.astype(jnp.bfloat16), jnp.float32).reshape(N_LEVELS * T)
    enc_t = _encode(xs, ys, zs, tabp)
    return _mlp(enc_t, W1, W2)
